# Initial kernel scaffold; baseline (speedup 1.0000x reference)
import jax
import jax.numpy as jnp
from jax.experimental import pallas as pl


def kernel(kemb, pos, z, batch_index, W1, b1, W2, b2, W_out, ref_table):
    raise NotImplementedError("write your pallas kernel here")


# speedup vs baseline: 7.9048x; 7.9048x over previous
import numpy as np
import jax
import jax.numpy as jnp
from jax import lax
from jax.experimental import pallas as pl
from jax.experimental.pallas import tpu as pltpu

_MASSES = np.array([0.0,1.008,4.0026,6.94,9.0122,10.81,12.011,14.007,15.999,18.998,20.18,22.99,24.305,26.982,28.085,30.974,32.06,35.45,39.948,39.098,40.078,44.956,47.867,50.942,51.996,54.938,55.845,58.933,58.693,63.546,65.38,69.723,72.63,74.922,78.971,79.904,83.798,85.468,87.62,88.906,91.224,92.906,95.95,97.907,101.07,102.906,106.42,107.868,112.414,114.818,118.71,121.76,127.6,126.904,131.293,132.905,137.327,138.905,140.116,140.908,144.242,144.913,150.36,151.964,157.25,158.925,162.5,164.93,167.259,168.934,173.054,174.967,178.49,180.948,183.84,186.207,190.23,192.217,195.084,196.967,200.592,204.38,207.2,208.98,208.982,209.987,222.018,223.02,226.025,227.028,232.038,231.036,238.029,237.048,244.064,243.061,247.07,247.07,251.08,252.083], dtype=np.float32)

_B = 16
_NZ = 100
_R = 512


def _sigmoid(x):
    return 1.0 / (1.0 + jnp.exp(-x))


def _softplus(x):
    return jnp.maximum(x, 0.0) + jnp.log(1.0 + jnp.exp(-jnp.abs(x)))


def _block(x_ref, aux_ref, w1_ref, b1_ref, w2_ref, b2_ref, wo_ref, tab_ref,
           out_ref):
    x = x_ref[...]
    h = jnp.dot(x, w1_ref[...], preferred_element_type=jnp.float32) + b1_ref[...]
    h = h * _sigmoid(h)
    h = jnp.dot(h, w2_ref[...], preferred_element_type=jnp.float32) + b2_ref[...]
    h = h * _sigmoid(h)
    q0 = jnp.dot(x + h, wo_ref[...], preferred_element_type=jnp.float32)

    aux = aux_ref[...]
    posb = aux[:, 0:3]
    zf = aux[:, 3:4]
    bf = aux[:, 4:5]
    rows = aux.shape[0]

    zoh = (zf == lax.broadcasted_iota(jnp.int32, (rows, _NZ), 1
                                      ).astype(jnp.float32)
           ).astype(jnp.float32)
    rm = jnp.dot(zoh, tab_ref[...], preferred_element_type=jnp.float32)
    q = _softplus(q0 + rm[:, 0:1])
    mass = rm[:, 1:2]

    r2 = jnp.sum(posb * posb, axis=1, keepdims=True)
    ones = jnp.ones_like(r2)
    u = jnp.concatenate([posb, r2, ones], axis=1)
    s = jnp.concatenate([q * u, mass * u, u], axis=1)

    soh = (bf == lax.broadcasted_iota(jnp.int32, (rows, _B), 1
                                      ).astype(jnp.float32)
           ).astype(jnp.float32)
    part = lax.dot_general(soh, s, (((0,), (0,)), ((), ())),
                           preferred_element_type=jnp.float32)

    @pl.when(pl.program_id(0) == 0)
    def _init():
        out_ref[...] = jnp.zeros_like(out_ref)

    out_ref[...] += part


def kernel(kemb, pos, z, batch_index, W1, b1, W2, b2, W_out, ref_table):
    n, h = kemb.shape
    aux = jnp.concatenate(
        [pos, z.astype(jnp.float32)[:, None],
         batch_index.astype(jnp.float32)[:, None]], axis=1)
    ref0 = ref_table.at[0].set(0.0)
    tab = jnp.concatenate([ref0, jnp.asarray(_MASSES)[:, None]], axis=1)

    sums = pl.pallas_call(
        _block,
        grid=(n // _R,),
        in_specs=[
            pl.BlockSpec((_R, h), lambda i: (i, 0)),
            pl.BlockSpec((_R, 5), lambda i: (i, 0)),
            pl.BlockSpec((h, h), lambda i: (0, 0)),
            pl.BlockSpec((1, h), lambda i: (0, 0)),
            pl.BlockSpec((h, h), lambda i: (0, 0)),
            pl.BlockSpec((1, h), lambda i: (0, 0)),
            pl.BlockSpec((h, 1), lambda i: (0, 0)),
            pl.BlockSpec((_NZ, 2), lambda i: (0, 0)),
        ],
        out_specs=pl.BlockSpec((_B, 15), lambda i: (0, 0)),
        out_shape=jax.ShapeDtypeStruct((_B, 15), jnp.float32),
        compiler_params=pltpu.CompilerParams(
            dimension_semantics=("arbitrary",)),
    )(kemb, aux, W1, b1[None, :], W2, b2[None, :], W_out, tab)

    t1 = sums[:, 0:3]
    t2 = sums[:, 3]
    t0 = sums[:, 4]
    s1 = sums[:, 5:8]
    s0 = sums[:, 9]
    cnt = sums[:, 14]
    com = s1 / s0[:, None]
    res = (t2 - 2.0 * jnp.sum(com * t1, axis=1)
           + jnp.sum(com * com, axis=1) * t0)
    return jnp.where(cnt > 0, res, 0.0)


# speedup vs baseline: 10.5488x; 1.3345x over previous
import functools

import numpy as np
import jax
import jax.numpy as jnp
from jax import lax
from jax.experimental import pallas as pl
from jax.experimental.pallas import tpu as pltpu
from jax.experimental.pallas import tpu_sc as plsc

_MASSES = np.array([0.0,1.008,4.0026,6.94,9.0122,10.81,12.011,14.007,15.999,18.998,20.18,22.99,24.305,26.982,28.085,30.974,32.06,35.45,39.948,39.098,40.078,44.956,47.867,50.942,51.996,54.938,55.845,58.933,58.693,63.546,65.38,69.723,72.63,74.922,78.971,79.904,83.798,85.468,87.62,88.906,91.224,92.906,95.95,97.907,101.07,102.906,106.42,107.868,112.414,114.818,118.71,121.76,127.6,126.904,131.293,132.905,137.327,138.905,140.116,140.908,144.242,144.913,150.36,151.964,157.25,158.925,162.5,164.93,167.259,168.934,173.054,174.967,178.49,180.948,183.84,186.207,190.23,192.217,195.084,196.967,200.592,204.38,207.2,208.98,208.982,209.987,222.018,223.02,226.025,227.028,232.038,231.036,238.029,237.048,244.064,243.061,247.07,247.07,251.08,252.083], dtype=np.float32)

_B = 16
_NZ = 100
_R = 512
_L = 16
_NW = 32


def _sigmoid(x):
    return 1.0 / (1.0 + jnp.exp(-x))


def _softplus(x):
    return jnp.maximum(x, 0.0) + jnp.log(1.0 + jnp.exp(-jnp.abs(x)))


def _tc_block(x_ref, aux_ref, w1_ref, b1_ref, w2_ref, b2_ref, wo_ref,
              tab_ref, out_ref):
    x = x_ref[...]
    h = jnp.dot(x, w1_ref[...], preferred_element_type=jnp.float32) + b1_ref[...]
    h = h * _sigmoid(h)
    h = jnp.dot(h, w2_ref[...], preferred_element_type=jnp.float32) + b2_ref[...]
    h = h * _sigmoid(h)
    q0 = jnp.dot(x + h, wo_ref[...], preferred_element_type=jnp.float32)

    aux = aux_ref[...]
    posb = aux[:, 0:3]
    zf = aux[:, 3:4]
    bf = aux[:, 4:5]
    rows = aux.shape[0]

    zoh = (zf == lax.broadcasted_iota(jnp.int32, (rows, _NZ), 1
                                      ).astype(jnp.float32)
           ).astype(jnp.float32)
    refz = jnp.dot(zoh, tab_ref[...], preferred_element_type=jnp.float32)
    q = _softplus(q0 + refz)

    r2 = jnp.sum(posb * posb, axis=1, keepdims=True)
    ones = jnp.ones_like(r2)
    u = jnp.concatenate([posb, r2, ones], axis=1)

    soh = (bf == lax.broadcasted_iota(jnp.int32, (rows, _B), 1
                                      ).astype(jnp.float32)
           ).astype(jnp.float32)
    part = lax.dot_general(soh, q * u, (((0,), (0,)), ((), ())),
                           preferred_element_type=jnp.float32)

    @pl.when(pl.program_id(0) == 0)
    def _init():
        out_ref[...] = jnp.zeros_like(out_ref)

    out_ref[...] += part


def _sc_stats_body(px_hbm, py_hbm, pz_hbm, z_hbm, b_hbm, tab_hbm, out_hbm,
                   px_v, py_v, pz_v, z_v, b_v, tab_v, acc_v, tot_v):
    chunk = px_v.shape[0]
    wid = lax.axis_index("s") * 2 + lax.axis_index("c")
    base = wid * chunk
    pltpu.sync_copy(px_hbm.at[pl.ds(base, chunk)], px_v)
    pltpu.sync_copy(py_hbm.at[pl.ds(base, chunk)], py_v)
    pltpu.sync_copy(pz_hbm.at[pl.ds(base, chunk)], pz_v)
    pltpu.sync_copy(z_hbm.at[pl.ds(base, chunk)], z_v)
    pltpu.sync_copy(b_hbm.at[pl.ds(base, chunk)], b_v)
    pltpu.sync_copy(tab_hbm, tab_v)

    zeros = jnp.zeros((_L,), jnp.float32)
    for k in range(5):
        for j in range(_L):
            acc_v[k, pl.ds(j * _L, _L)] = zeros

    lane = lax.iota(jnp.int32, _L)
    ones = jnp.ones((_L,), jnp.float32)

    def body(i, carry):
        off = i * _L
        zv = z_v[pl.ds(off, _L)]
        bv = b_v[pl.ds(off, _L)]
        pxv = px_v[pl.ds(off, _L)]
        pyv = py_v[pl.ds(off, _L)]
        pzv = pz_v[pl.ds(off, _L)]
        m = plsc.load_gather(tab_v, [zv])
        vidx = lane * _L + bv
        for k, val in ((0, ones), (1, m), (2, m * pxv), (3, m * pyv),
                       (4, m * pzv)):
            plsc.addupdate_scatter(
                acc_v, [jnp.full((_L,), k, jnp.int32), vidx], val)
        return carry

    lax.fori_loop(0, chunk // _L, body, 0)

    for k in range(5):
        tot = acc_v[k, pl.ds(0, _L)]
        for j in range(1, _L):
            tot = tot + acc_v[k, pl.ds(j * _L, _L)]
        tot_v[k, :] = tot
    pltpu.sync_copy(tot_v, out_hbm.at[wid])


def _sc_stats(px, py, pz, z, b, tab):
    n = px.shape[0]
    chunk = n // _NW
    mesh = plsc.VectorSubcoreMesh(core_axis_name="c", subcore_axis_name="s",
                                  num_cores=2, num_subcores=16)
    return pl.kernel(
        _sc_stats_body,
        out_type=jax.ShapeDtypeStruct((_NW, 5, _L), jnp.float32),
        mesh=mesh,
        compiler_params=pltpu.CompilerParams(needs_layout_passes=False),
        scratch_types=[
            pltpu.VMEM((chunk,), jnp.float32),
            pltpu.VMEM((chunk,), jnp.float32),
            pltpu.VMEM((chunk,), jnp.float32),
            pltpu.VMEM((chunk,), jnp.int32),
            pltpu.VMEM((chunk,), jnp.int32),
            pltpu.VMEM((128,), jnp.float32),
            pltpu.VMEM((5, _L * _L), jnp.float32),
            pltpu.VMEM((5, _L), jnp.float32),
        ],
    )(px, py, pz, z, b, tab)


def kernel(kemb, pos, z, batch_index, W1, b1, W2, b2, W_out, ref_table):
    n, h = kemb.shape
    zi = z.astype(jnp.int32)
    bi = batch_index.astype(jnp.int32)
    aux = jnp.concatenate(
        [pos, zi.astype(jnp.float32)[:, None],
         bi.astype(jnp.float32)[:, None]], axis=1)
    ref0 = ref_table.at[0].set(0.0)
    mass_tab = jnp.pad(jnp.asarray(_MASSES), (0, 28))

    sc_part = _sc_stats(pos[:, 0], pos[:, 1], pos[:, 2], zi, bi, mass_tab)

    tsums = pl.pallas_call(
        _tc_block,
        grid=(n // _R,),
        in_specs=[
            pl.BlockSpec((_R, h), lambda i: (i, 0)),
            pl.BlockSpec((_R, 5), lambda i: (i, 0)),
            pl.BlockSpec((h, h), lambda i: (0, 0)),
            pl.BlockSpec((1, h), lambda i: (0, 0)),
            pl.BlockSpec((h, h), lambda i: (0, 0)),
            pl.BlockSpec((1, h), lambda i: (0, 0)),
            pl.BlockSpec((h, 1), lambda i: (0, 0)),
            pl.BlockSpec((_NZ, 1), lambda i: (0, 0)),
        ],
        out_specs=pl.BlockSpec((_B, 5), lambda i: (0, 0)),
        out_shape=jax.ShapeDtypeStruct((_B, 5), jnp.float32),
        compiler_params=pltpu.CompilerParams(
            dimension_semantics=("arbitrary",)),
    )(kemb, aux, W1, b1[None, :], W2, b2[None, :], W_out, ref0)

    sc = jnp.sum(sc_part, axis=0)
    cnt = sc[0]
    s0 = sc[1]
    s1 = sc[2:5]
    t1 = tsums[:, 0:3]
    t2 = tsums[:, 3]
    t0 = tsums[:, 4]
    com = s1 / s0
    res = (t2 - 2.0 * jnp.sum(com.T * t1, axis=1)
           + jnp.sum(com * com, axis=0) * t0)
    return jnp.where(cnt > 0, res, 0.0)
